# baseline (speedup 1.0000x reference)
import jax
import jax.numpy as jnp
from jax import lax
from jax.experimental import pallas as pl
from jax.experimental.pallas import tpu as pltpu

B = 16
N = 10000
H = 128
F = 4
IN = 2 * H + 2 * F
G3 = 3 * H


def _body(uid_ref, iid_ref, uf_ref, itf_ref,
          wu_ref, bihu_ref, bhhu_ref,
          wi_ref, bihi_ref, bhhi_ref,
          ublk_ref, iblk_ref, umem_ref, imem_ref,
          nu_ref, ni_ref, uout_ref, iout_ref,
          ue_ref, ie_ref, xu_ref, xi_ref, sem_g):
    b = pl.program_id(0)

    @pl.when(b == 0)
    def _compute():
        gath = [pltpu.make_async_copy(umem_ref.at[k, uid_ref[k]], ue_ref.at[k],
                                      sem_g) for k in range(B)]
        gath += [pltpu.make_async_copy(imem_ref.at[k, iid_ref[k]], ie_ref.at[k],
                                       sem_g) for k in range(B)]
        for c in gath:
            c.start()
        for c in gath:
            c.wait()

        ue = ue_ref[...]
        ie = ie_ref[...]
        uf = uf_ref[...]
        itf = itf_ref[...]

        xu_ref[:, 0:H] = ue
        xu_ref[:, H:H + F] = uf
        xu_ref[:, H + F:H + F + H] = ie
        xu_ref[:, H + F + H:IN] = itf
        xi_ref[:, 0:H] = ie
        xi_ref[:, H:H + F] = itf
        xi_ref[:, H + F:H + F + H] = ue
        xi_ref[:, H + F + H:IN] = uf

        def gru(x_ref, w_ref, bih_ref, bhh_ref):
            gx = lax.dot_general(x_ref[...], w_ref[...],
                                 (((1,), (1,)), ((), ())),
                                 preferred_element_type=jnp.float32)
            gx = gx + bih_ref[...]
            bhh = bhh_ref[...]
            g = gx + bhh
            r = jax.nn.sigmoid(g[:, :H])
            z = jax.nn.sigmoid(g[:, H:2 * H])
            n = jnp.tanh(gx[:, 2 * H:] + r * bhh[:, 2 * H:])
            out = (1.0 - z) * n
            nrm = jnp.sqrt(jnp.sum(out * out, axis=1, keepdims=True))
            return out / jnp.maximum(nrm, 1e-12)

        nu_ref[...] = gru(xu_ref, wu_ref, bihu_ref, bhhu_ref)
        ni_ref[...] = gru(xi_ref, wi_ref, bihi_ref, bhhi_ref)

    uout_ref[...] = ublk_ref[...]
    iout_ref[...] = iblk_ref[...]

    uid = uid_ref[b]
    iid = iid_ref[b]
    uout_ref[0, pl.ds(uid, 1), :] = nu_ref[pl.ds(b, 1), :]
    iout_ref[0, pl.ds(iid, 1), :] = ni_ref[pl.ds(b, 1), :]


def kernel(user_ids, item_ids, user_features, item_features, user_memory,
           item_memory, W_ih_u, W_hh_u, b_ih_u, b_hh_u, W_ih_i, W_hh_i,
           b_ih_i, b_hh_i):
    del W_hh_u, W_hh_i
    vmem = pl.BlockSpec(memory_space=pltpu.VMEM)
    smem = pl.BlockSpec(memory_space=pltpu.SMEM)
    anym = pl.BlockSpec(memory_space=pltpu.MemorySpace.HBM)
    blk = pl.BlockSpec((1, N, H), lambda b: (b, 0, 0))
    f32 = jnp.float32
    return pl.pallas_call(
        _body,
        grid=(B,),
        out_shape=(
            jax.ShapeDtypeStruct((B, H), f32),
            jax.ShapeDtypeStruct((B, H), f32),
            jax.ShapeDtypeStruct((B, N, H), f32),
            jax.ShapeDtypeStruct((B, N, H), f32),
        ),
        in_specs=[smem, smem] + [vmem] * 8 + [blk, blk, anym, anym],
        out_specs=(
            pl.BlockSpec((B, H), lambda b: (0, 0)),
            pl.BlockSpec((B, H), lambda b: (0, 0)),
            blk,
            blk,
        ),
        scratch_shapes=[
            pltpu.VMEM((B, H), f32),
            pltpu.VMEM((B, H), f32),
            pltpu.VMEM((B, IN), f32),
            pltpu.VMEM((B, IN), f32),
            pltpu.SemaphoreType.DMA,
        ],
    )(user_ids, item_ids, user_features, item_features,
      W_ih_u, b_ih_u.reshape(1, G3), b_hh_u.reshape(1, G3),
      W_ih_i, b_ih_i.reshape(1, G3), b_hh_i.reshape(1, G3),
      user_memory, item_memory, user_memory, item_memory)
